# baseline (speedup 1.0000x reference)
import functools

import jax
import jax.numpy as jnp
from jax import lax
from jax.experimental import pallas as pl
from jax.experimental.pallas import tpu as pltpu
from jax.experimental.pallas import tpu_sc as plsc

_NC = 2
_NS = 16
_NW = _NC * _NS

_NBUF = 2
_GB = 128
_NG = 2
_BAT = _NG * _GB


def _body(xt_hbm, table_hbm, out_hbm, idx_v, rows_v, outt_v, *sems):
  n = xt_hbm.shape[0]
  d = out_hbm.shape[1] * 8
  nbat = n // _BAT // _NW
  nlap = nbat // _NBUF
  gsems = sems[:_NBUF]
  ssems = sems[_NBUF:]

  wid = lax.axis_index("s") * _NC + lax.axis_index("c")
  base_t = wid * nbat

  pltpu.sync_copy(xt_hbm.at[pl.ds(base_t * _BAT, nbat * _BAT)], idx_v)

  iota16 = lax.iota(jnp.int32, 16)
  dtv = [(iota16 >> 3) + 2 * k for k in range(d // 16)]
  div = iota16 & 7

  def gather(t, s):
    return pltpu.make_async_copy(
        table_hbm.at[idx_v.at[pl.ds(t * _BAT, _BAT)]], rows_v.at[s], gsems[s])

  def store(t, s):
    gw = (base_t + t) * _NG
    h = gw // 128
    b0 = gw % 128
    return [pltpu.make_async_copy(outt_v.at[s, :, dt, :, pl.ds(0, _GB)],
                                  out_hbm.at[h, dt, pl.ds(b0, _NG)],
                                  ssems[s])
            for dt in range(d // 8)]

  def transpose(s):
    for g in range(_NG):
      gv = jnp.full((16,), g, jnp.int32)

      @pl.loop(0, _GB, unroll=8)
      def _(row):
        biv = jnp.full((16,), row, jnp.int32)
        for k in range(d // 16):
          vals = rows_v[s, g * _GB + row, pl.ds(16 * k, 16)]
          plsc.store_scatter(outt_v.at[s], [gv, dtv[k], div, biv], vals)

  for s in range(_NBUF):
    gather(s, s).start()

  @pl.loop(0, nlap)
  def _(lap):
    t0 = lap * _NBUF
    for s in range(_NBUF):
      t = t0 + s
      gather(t, s).wait()

      @pl.when(lap > 0)
      def _():
        for cp in store(t - _NBUF, s):
          cp.wait()

      transpose(s)
      for cp in store(t, s):
        cp.start()

      @pl.when(lap < nlap - 1)
      def _():
        gather(t + _NBUF, s).start()

  for s in range(_NBUF):
    for cp in store(nbat - _NBUF + s, s):
      cp.wait()


def _run(xt_flat, table, d):
  n = xt_flat.shape[0]
  nh = n // 16384
  per_w = n // _NW
  mesh = plsc.VectorSubcoreMesh(core_axis_name="c", subcore_axis_name="s")
  sems = [pltpu.SemaphoreType.DMA] * (2 * _NBUF)
  return pl.kernel(
      _body,
      out_type=jax.ShapeDtypeStruct((nh, d // 8, 128, 8, 128), table.dtype),
      mesh=mesh,
      compiler_params=pltpu.CompilerParams(use_tc_tiling_on_sc=False,
                                           needs_layout_passes=False),
      scratch_types=[
          pltpu.VMEM((per_w,), jnp.int32),
          pltpu.VMEM((_NBUF, _BAT, table.shape[1]), table.dtype),
          pltpu.VMEM((_NBUF, _NG, d // 8, 8, 129), table.dtype),
      ] + sems,
  )(xt_flat, table)


@jax.jit
def kernel(x, table):
  b, h = x.shape
  d = table.shape[1]
  xt = jnp.transpose(x).reshape(b * h).astype(jnp.int32)
  out5 = _run(xt, table, d)
  return jnp.transpose(out5, (2, 4, 0, 1, 3)).reshape(b, h, d)
